# baseline (speedup 1.0000x reference)
import jax
import jax.numpy as jnp
from jax.experimental import pallas as pl
from jax.experimental.pallas import tpu as pltpu

_SMOOTHING = 0.1
_N_CLASSES = 1000
_A = _SMOOTHING / (_N_CLASSES - 1)
_B = (1.0 - _SMOOTHING) - _A

_N_SAMPLES = 16384
_ROWS_PER_STRIPE = 40
_CCHUNK = 2048
_INV_N_ROWS = 1.0 / _N_SAMPLES


def _body(x_ref, t_ref, out_ref, m8s, s8s, sx8s, xt8s):
    i = pl.program_id(0)
    base = i * _ROWS_PER_STRIPE
    nk = _ROWS_PER_STRIPE // 8

    @pl.when(i == 0)
    def _init():
        m8s[...] = jnp.full((8, _N_SAMPLES), -jnp.inf, jnp.float32)
        s8s[...] = jnp.zeros((8, _N_SAMPLES), jnp.float32)
        sx8s[...] = jnp.zeros((8, _N_SAMPLES), jnp.float32)
        xt8s[...] = jnp.zeros((8, _N_SAMPLES), jnp.float32)

    row8 = jax.lax.broadcasted_iota(jnp.int32, (8, _CCHUNK), 0)
    for j in range(_N_SAMPLES // _CCHUNK):
        sl = pl.ds(j * _CCHUNK, _CCHUNK)

        m_loc = x_ref[0:8, sl]
        for k in range(1, nk):
            m_loc = jnp.maximum(m_loc, x_ref[k * 8:(k + 1) * 8, sl])
        m_old = m8s[:, sl]
        m_new = jnp.maximum(m_old, m_loc)
        s8 = s8s[:, sl] * jnp.exp(m_old - m_new)
        sx8 = sx8s[:, sl]
        xt8 = xt8s[:, sl]

        tb = jnp.broadcast_to(t_ref[sl][None, :] - base, (8, _CCHUNK))
        tb8 = tb - row8
        for k in range(nk):
            c = x_ref[k * 8:(k + 1) * 8, sl]
            s8 = s8 + jnp.exp(c - m_new)
            sx8 = sx8 + c
            eq = tb8 == (k * 8)
            xt8 = xt8 + jnp.where(eq, c, 0.0)

        m8s[:, sl] = m_new
        s8s[:, sl] = s8
        sx8s[:, sl] = sx8
        xt8s[:, sl] = xt8

    @pl.when(i == pl.num_programs(0) - 1)
    def _fin():
        m8 = m8s[...]
        mf = jnp.max(m8, axis=0, keepdims=True)
        sf = jnp.sum(s8s[...] * jnp.exp(m8 - mf), axis=0)
        lse = mf[0] + jnp.log(sf)
        sx1 = jnp.sum(sx8s[...], axis=0)
        xt1 = jnp.sum(xt8s[...], axis=0)
        out_ref[0, 0] = jnp.sum(lse - _A * sx1 - _B * xt1) * _INV_N_ROWS


def kernel(pred, target):
    predt = pred.T
    grid = _N_CLASSES // _ROWS_PER_STRIPE
    total = pl.pallas_call(
        _body,
        grid=(grid,),
        in_specs=[
            pl.BlockSpec((_ROWS_PER_STRIPE, _N_SAMPLES), lambda i: (i, 0)),
            pl.BlockSpec((_N_SAMPLES,), lambda i: (0,)),
        ],
        out_specs=pl.BlockSpec((1, 1), lambda i: (0, 0), memory_space=pltpu.SMEM),
        out_shape=jax.ShapeDtypeStruct((1, 1), jnp.float32),
        scratch_shapes=[
            pltpu.VMEM((8, _N_SAMPLES), jnp.float32),
            pltpu.VMEM((8, _N_SAMPLES), jnp.float32),
            pltpu.VMEM((8, _N_SAMPLES), jnp.float32),
            pltpu.VMEM((8, _N_SAMPLES), jnp.float32),
        ],
    )(predt, target.astype(jnp.int32))
    return total[0, 0]


# speedup vs baseline: 1.3085x; 1.3085x over previous
import jax
import jax.numpy as jnp
from jax.experimental import pallas as pl
from jax.experimental.pallas import tpu as pltpu

_SMOOTHING = 0.1
_N_CLASSES = 1000
_A = _SMOOTHING / (_N_CLASSES - 1)
_B = (1.0 - _SMOOTHING) - _A

_N_SAMPLES = 16384
_ROWS_PER_STRIPE = 200
_CCHUNK = 2048
_INV_N_ROWS = 1.0 / _N_SAMPLES


def _body(x_ref, t_ref, out_ref, m8s, s8s, sx8s, xt8s):
    i = pl.program_id(0)
    base = i * _ROWS_PER_STRIPE
    nk = _ROWS_PER_STRIPE // 8

    @pl.when(i == 0)
    def _init():
        m8s[...] = jnp.full((8, _N_SAMPLES), -jnp.inf, jnp.float32)
        s8s[...] = jnp.zeros((8, _N_SAMPLES), jnp.float32)
        sx8s[...] = jnp.zeros((8, _N_SAMPLES), jnp.float32)
        xt8s[...] = jnp.zeros((8, _N_SAMPLES), jnp.float32)

    row8 = jax.lax.broadcasted_iota(jnp.int32, (8, _CCHUNK), 0)
    for j in range(_N_SAMPLES // _CCHUNK):
        sl = pl.ds(j * _CCHUNK, _CCHUNK)

        m_loc = x_ref[0:8, sl]
        for k in range(1, nk):
            m_loc = jnp.maximum(m_loc, x_ref[k * 8:(k + 1) * 8, sl])
        m_old = m8s[:, sl]
        m_new = jnp.maximum(m_old, m_loc)
        s8 = s8s[:, sl] * jnp.exp(m_old - m_new)
        sx8 = sx8s[:, sl]
        xt8 = xt8s[:, sl]

        tb = jnp.broadcast_to(t_ref[sl][None, :] - base, (8, _CCHUNK))
        tb8 = tb - row8
        for k in range(nk):
            c = x_ref[k * 8:(k + 1) * 8, sl]
            s8 = s8 + jnp.exp(c - m_new)
            sx8 = sx8 + c
            eq = tb8 == (k * 8)
            xt8 = xt8 + jnp.where(eq, c, 0.0)

        m8s[:, sl] = m_new
        s8s[:, sl] = s8
        sx8s[:, sl] = sx8
        xt8s[:, sl] = xt8

    @pl.when(i == pl.num_programs(0) - 1)
    def _fin():
        m8 = m8s[...]
        mf = jnp.max(m8, axis=0, keepdims=True)
        sf = jnp.sum(s8s[...] * jnp.exp(m8 - mf), axis=0)
        lse = mf[0] + jnp.log(sf)
        sx1 = jnp.sum(sx8s[...], axis=0)
        xt1 = jnp.sum(xt8s[...], axis=0)
        out_ref[0, 0] = jnp.sum(lse - _A * sx1 - _B * xt1) * _INV_N_ROWS


def kernel(pred, target):
    predt = pred.T
    grid = _N_CLASSES // _ROWS_PER_STRIPE
    total = pl.pallas_call(
        _body,
        grid=(grid,),
        in_specs=[
            pl.BlockSpec((_ROWS_PER_STRIPE, _N_SAMPLES), lambda i: (i, 0)),
            pl.BlockSpec((_N_SAMPLES,), lambda i: (0,)),
        ],
        out_specs=pl.BlockSpec((1, 1), lambda i: (0, 0), memory_space=pltpu.SMEM),
        out_shape=jax.ShapeDtypeStruct((1, 1), jnp.float32),
        scratch_shapes=[
            pltpu.VMEM((8, _N_SAMPLES), jnp.float32),
            pltpu.VMEM((8, _N_SAMPLES), jnp.float32),
            pltpu.VMEM((8, _N_SAMPLES), jnp.float32),
            pltpu.VMEM((8, _N_SAMPLES), jnp.float32),
        ],
    )(predt, target.astype(jnp.int32))
    return total[0, 0]
